# baseline (device time: 100136 ns/iter reference)
import jax
import jax.numpy as jnp
from jax import lax
from jax.experimental import pallas as pl
from jax.experimental.pallas import tpu as pltpu

N_DEV = 32
NSEG = 8
NBUF = 4


def _snake_coords():
    coords = []
    for z in range(4):
        for y in range(4):
            xs = (0, 1) if y % 2 == 0 else (1, 0)
            for x in xs:
                coords.append((x, y, z))
    return coords


_C16 = [(0, 0), (0, 1), (0, 2), (0, 3), (1, 3), (1, 2), (1, 1), (2, 1),
        (2, 2), (2, 3), (3, 3), (3, 2), (3, 1), (3, 0), (2, 0), (1, 0)]


def _ham_ring():
    ring = []
    for k, (y, z) in enumerate(_C16):
        if k % 2 == 0:
            ring += [(0, y, z), (1, y, z)]
        else:
            ring += [(1, y, z), (0, y, z)]
    return ring


_RING = _ham_ring()
for _a, _b in zip(_RING, _RING[1:] + _RING[:1]):
    assert sum(abs(p - q) for p, q in zip(_a, _b)) == 1, (_a, _b)

_POS_OF = {c: p for p, c in enumerate(_snake_coords())}
RING_TO_POS = [_POS_OF[c] for c in _RING]
POS_TO_RING = [0] * N_DEV
for _r, _p in enumerate(RING_TO_POS):
    POS_TO_RING[_p] = _r


def kernel(x, w_mat):
    m_global, k_per = x.shape
    _, n = w_mat.shape
    m_per = m_global // N_DEV
    half = n // 2
    segw = half // NSEG

    r2p = jnp.asarray(RING_TO_POS, dtype=jnp.int32)
    p2r = jnp.asarray(POS_TO_RING, dtype=jnp.int32)

    def body(r2p_ref, p2r_ref, x_ref, w_ref, out_ref, buf_r, buf_l,
             send_r, recv_r, send_l, recv_l, cred_r, cred_l):
        my = lax.axis_index("i")
        k = p2r_ref[my]

        def ring_pos(delta):
            return r2p_ref[lax.rem(k + delta + 2 * N_DEV, N_DEV)]

        right = ring_pos(1)
        left = ring_pos(-1)

        barrier_sem = pltpu.get_barrier_semaphore()
        for nbr in (left, right):
            pl.semaphore_signal(
                barrier_sem, inc=1,
                device_id=(nbr,), device_id_type=pl.DeviceIdType.MESH,
            )
        pl.semaphore_wait(barrier_sem, 2)

        def partial(c, col0):
            xb = x_ref[pl.ds(c * m_per, m_per), :]
            return jnp.dot(
                xb, w_ref[:, col0:col0 + half],
                preferred_element_type=jnp.float32,
            )

        def mk(buf, ssem, rsem, j4, s, dev):
            return pltpu.make_async_remote_copy(
                src_ref=buf.at[j4, :, s * segw:(s + 1) * segw],
                dst_ref=buf.at[(j4 + 1) % NBUF, :, s * segw:(s + 1) * segw],
                send_sem=ssem.at[s, j4 % 2],
                recv_sem=rsem.at[s, j4 % 2],
                device_id=(dev,),
                device_id_type=pl.DeviceIdType.MESH,
            )

        def hop(h, h4, mid, is_last, credit=True):
            p_r = partial(ring_pos(-2 - h), 0)
            p_l = partial(ring_pos(2 + h), half)
            d4 = (h4 + 1) % NBUF

            if mid:
                for s in range(NSEG):
                    mk(buf_r, send_r, recv_r, (h4 + 3) % NBUF, s,
                       right).wait_send()
                    mk(buf_l, send_l, recv_l, (h4 + 3) % NBUF, s,
                       left).wait_send()
                pl.semaphore_wait(cred_r.at[(h4 + 1) % 2], 1)
                pl.semaphore_wait(cred_l.at[(h4 + 1) % 2], 1)

            for s in range(NSEG):
                cols = slice(s * segw, (s + 1) * segw)

                mk(buf_r, send_r, recv_r, h4, s, right).wait_recv()
                if not is_last:
                    buf_r[d4, :, cols] = (
                        buf_r[d4, :, cols].astype(jnp.float32)
                        + p_r[:, cols]
                    ).astype(jnp.bfloat16)
                    mk(buf_r, send_r, recv_r, d4, s, right).start()
                else:
                    out_ref[:, cols] = (
                        buf_r[d4, :, cols].astype(jnp.float32)
                        + p_r[:, cols]
                    )

                mk(buf_l, send_l, recv_l, h4, s, left).wait_recv()
                if not is_last:
                    buf_l[d4, :, cols] = (
                        buf_l[d4, :, cols].astype(jnp.float32)
                        + p_l[:, cols]
                    ).astype(jnp.bfloat16)
                    mk(buf_l, send_l, recv_l, d4, s, left).start()
                else:
                    out_ref[:, half + s * segw:half + (s + 1) * segw] = (
                        buf_l[d4, :, cols].astype(jnp.float32)
                        + p_l[:, cols]
                    )

            if credit:
                pl.semaphore_signal(
                    cred_r.at[h4 % 2], inc=1,
                    device_id=(left,),
                    device_id_type=pl.DeviceIdType.MESH,
                )
                pl.semaphore_signal(
                    cred_l.at[h4 % 2], inc=1,
                    device_id=(right,),
                    device_id_type=pl.DeviceIdType.MESH,
                )

        buf_r[0, :, :] = partial(ring_pos(-1), 0).astype(jnp.bfloat16)
        buf_l[0, :, :] = partial(ring_pos(1), half).astype(jnp.bfloat16)
        for s in range(NSEG):
            mk(buf_r, send_r, recv_r, 0, s, right).start()
            mk(buf_l, send_l, recv_l, 0, s, left).start()

        hop(0, 0, mid=False, is_last=False)

        def four_hops(t, carry):
            h = 4 * t + 1
            hop(h, 1, mid=True, is_last=False)
            hop(h + 1, 2, mid=True, is_last=False)
            hop(h + 2, 3, mid=True, is_last=False)
            hop(h + 3, 0, mid=True, is_last=False)
            return carry

        lax.fori_loop(0, (N_DEV - 4) // NBUF, four_hops, 0)

        hop(N_DEV - 3, (N_DEV - 3) % NBUF, mid=True, is_last=False,
            credit=False)
        hop(N_DEV - 2, (N_DEV - 2) % NBUF, mid=False, is_last=True,
            credit=False)

        for s in range(NSEG):
            mk(buf_r, send_r, recv_r, (N_DEV - 3) % NBUF, s, right).wait_send()
            mk(buf_l, send_l, recv_l, (N_DEV - 3) % NBUF, s, left).wait_send()
            mk(buf_r, send_r, recv_r, (N_DEV - 2) % NBUF, s, right).wait_send()
            mk(buf_l, send_l, recv_l, (N_DEV - 2) % NBUF, s, left).wait_send()

    return pl.pallas_call(
        body,
        out_shape=jax.ShapeDtypeStruct((m_per, n), jnp.float32),
        in_specs=[
            pl.BlockSpec(memory_space=pltpu.SMEM),
            pl.BlockSpec(memory_space=pltpu.SMEM),
            pl.BlockSpec(memory_space=pltpu.VMEM),
            pl.BlockSpec(memory_space=pltpu.VMEM),
        ],
        out_specs=pl.BlockSpec(memory_space=pltpu.VMEM),
        scratch_shapes=[
            pltpu.VMEM((NBUF, m_per, half), jnp.bfloat16),
            pltpu.VMEM((NBUF, m_per, half), jnp.bfloat16),
            pltpu.SemaphoreType.DMA((NSEG, 2)),
            pltpu.SemaphoreType.DMA((NSEG, 2)),
            pltpu.SemaphoreType.DMA((NSEG, 2)),
            pltpu.SemaphoreType.DMA((NSEG, 2)),
            pltpu.SemaphoreType.REGULAR((2,)),
            pltpu.SemaphoreType.REGULAR((2,)),
        ],
        compiler_params=pltpu.CompilerParams(collective_id=0),
    )(r2p, p2r, x, w_mat)


# device time: 99941 ns/iter; 1.0020x vs baseline; 1.0020x over previous
import jax
import jax.numpy as jnp
from jax import lax
from jax.experimental import pallas as pl
from jax.experimental.pallas import tpu as pltpu

N_DEV = 32
NSEG = 4
NBUF = 4


def _snake_coords():
    coords = []
    for z in range(4):
        for y in range(4):
            xs = (0, 1) if y % 2 == 0 else (1, 0)
            for x in xs:
                coords.append((x, y, z))
    return coords


_C16 = [(0, 0), (0, 1), (0, 2), (0, 3), (1, 3), (1, 2), (1, 1), (2, 1),
        (2, 2), (2, 3), (3, 3), (3, 2), (3, 1), (3, 0), (2, 0), (1, 0)]


def _ham_ring():
    ring = []
    for k, (y, z) in enumerate(_C16):
        if k % 2 == 0:
            ring += [(0, y, z), (1, y, z)]
        else:
            ring += [(1, y, z), (0, y, z)]
    return ring


_RING = _ham_ring()
for _a, _b in zip(_RING, _RING[1:] + _RING[:1]):
    assert sum(abs(p - q) for p, q in zip(_a, _b)) == 1, (_a, _b)

_POS_OF = {c: p for p, c in enumerate(_snake_coords())}
RING_TO_POS = [_POS_OF[c] for c in _RING]
POS_TO_RING = [0] * N_DEV
for _r, _p in enumerate(RING_TO_POS):
    POS_TO_RING[_p] = _r


def kernel(x, w_mat):
    m_global, k_per = x.shape
    _, n = w_mat.shape
    m_per = m_global // N_DEV
    half = n // 2
    segw = half // NSEG

    r2p = jnp.asarray(RING_TO_POS, dtype=jnp.int32)
    p2r = jnp.asarray(POS_TO_RING, dtype=jnp.int32)

    def body(r2p_ref, p2r_ref, x_ref, w_ref, out_ref, buf_r, buf_l,
             send_r, recv_r, send_l, recv_l, cred_r, cred_l):
        my = lax.axis_index("i")
        k = p2r_ref[my]

        def ring_pos(delta):
            return r2p_ref[lax.rem(k + delta + 2 * N_DEV, N_DEV)]

        right = ring_pos(1)
        left = ring_pos(-1)

        barrier_sem = pltpu.get_barrier_semaphore()
        for nbr in (left, right):
            pl.semaphore_signal(
                barrier_sem, inc=1,
                device_id=(nbr,), device_id_type=pl.DeviceIdType.MESH,
            )
        pl.semaphore_wait(barrier_sem, 2)

        def partial(c, col0):
            xb = x_ref[pl.ds(c * m_per, m_per), :]
            return jnp.dot(
                xb, w_ref[:, col0:col0 + half],
                preferred_element_type=jnp.float32,
            )

        def mk(buf, ssem, rsem, j4, s, dev):
            return pltpu.make_async_remote_copy(
                src_ref=buf.at[j4, :, s * segw:(s + 1) * segw],
                dst_ref=buf.at[(j4 + 1) % NBUF, :, s * segw:(s + 1) * segw],
                send_sem=ssem.at[s, j4 % 2],
                recv_sem=rsem.at[s, j4 % 2],
                device_id=(dev,),
                device_id_type=pl.DeviceIdType.MESH,
            )

        def hop(h, h4, mid, is_last, credit=True):
            p_r = partial(ring_pos(-2 - h), 0)
            p_l = partial(ring_pos(2 + h), half)
            d4 = (h4 + 1) % NBUF

            if mid:
                for s in range(NSEG):
                    mk(buf_r, send_r, recv_r, (h4 + 3) % NBUF, s,
                       right).wait_send()
                    mk(buf_l, send_l, recv_l, (h4 + 3) % NBUF, s,
                       left).wait_send()
                pl.semaphore_wait(cred_r.at[(h4 + 1) % 2], 1)
                pl.semaphore_wait(cred_l.at[(h4 + 1) % 2], 1)

            for s in range(NSEG):
                cols = slice(s * segw, (s + 1) * segw)

                mk(buf_r, send_r, recv_r, h4, s, right).wait_recv()
                if not is_last:
                    buf_r[d4, :, cols] = (
                        buf_r[d4, :, cols].astype(jnp.float32)
                        + p_r[:, cols]
                    ).astype(jnp.bfloat16)
                    mk(buf_r, send_r, recv_r, d4, s, right).start()
                else:
                    out_ref[:, cols] = (
                        buf_r[d4, :, cols].astype(jnp.float32)
                        + p_r[:, cols]
                    )

                mk(buf_l, send_l, recv_l, h4, s, left).wait_recv()
                if not is_last:
                    buf_l[d4, :, cols] = (
                        buf_l[d4, :, cols].astype(jnp.float32)
                        + p_l[:, cols]
                    ).astype(jnp.bfloat16)
                    mk(buf_l, send_l, recv_l, d4, s, left).start()
                else:
                    out_ref[:, half + s * segw:half + (s + 1) * segw] = (
                        buf_l[d4, :, cols].astype(jnp.float32)
                        + p_l[:, cols]
                    )

            if credit:
                pl.semaphore_signal(
                    cred_r.at[h4 % 2], inc=1,
                    device_id=(left,),
                    device_id_type=pl.DeviceIdType.MESH,
                )
                pl.semaphore_signal(
                    cred_l.at[h4 % 2], inc=1,
                    device_id=(right,),
                    device_id_type=pl.DeviceIdType.MESH,
                )

        buf_r[0, :, :] = partial(ring_pos(-1), 0).astype(jnp.bfloat16)
        buf_l[0, :, :] = partial(ring_pos(1), half).astype(jnp.bfloat16)
        for s in range(NSEG):
            mk(buf_r, send_r, recv_r, 0, s, right).start()
            mk(buf_l, send_l, recv_l, 0, s, left).start()

        hop(0, 0, mid=False, is_last=False)

        def four_hops(t, carry):
            h = 4 * t + 1
            hop(h, 1, mid=True, is_last=False)
            hop(h + 1, 2, mid=True, is_last=False)
            hop(h + 2, 3, mid=True, is_last=False)
            hop(h + 3, 0, mid=True, is_last=False)
            return carry

        lax.fori_loop(0, (N_DEV - 4) // NBUF, four_hops, 0)

        hop(N_DEV - 3, (N_DEV - 3) % NBUF, mid=True, is_last=False,
            credit=False)
        hop(N_DEV - 2, (N_DEV - 2) % NBUF, mid=False, is_last=True,
            credit=False)

        for s in range(NSEG):
            mk(buf_r, send_r, recv_r, (N_DEV - 3) % NBUF, s, right).wait_send()
            mk(buf_l, send_l, recv_l, (N_DEV - 3) % NBUF, s, left).wait_send()
            mk(buf_r, send_r, recv_r, (N_DEV - 2) % NBUF, s, right).wait_send()
            mk(buf_l, send_l, recv_l, (N_DEV - 2) % NBUF, s, left).wait_send()

    return pl.pallas_call(
        body,
        out_shape=jax.ShapeDtypeStruct((m_per, n), jnp.float32),
        in_specs=[
            pl.BlockSpec(memory_space=pltpu.SMEM),
            pl.BlockSpec(memory_space=pltpu.SMEM),
            pl.BlockSpec(memory_space=pltpu.VMEM),
            pl.BlockSpec(memory_space=pltpu.VMEM),
        ],
        out_specs=pl.BlockSpec(memory_space=pltpu.VMEM),
        scratch_shapes=[
            pltpu.VMEM((NBUF, m_per, half), jnp.bfloat16),
            pltpu.VMEM((NBUF, m_per, half), jnp.bfloat16),
            pltpu.SemaphoreType.DMA((NSEG, 2)),
            pltpu.SemaphoreType.DMA((NSEG, 2)),
            pltpu.SemaphoreType.DMA((NSEG, 2)),
            pltpu.SemaphoreType.DMA((NSEG, 2)),
            pltpu.SemaphoreType.REGULAR((2,)),
            pltpu.SemaphoreType.REGULAR((2,)),
        ],
        compiler_params=pltpu.CompilerParams(collective_id=0),
    )(r2p, p2r, x, w_mat)
